# baseline (speedup 1.0000x reference)
import functools

import jax
import jax.numpy as jnp
from jax import lax
from jax.experimental import pallas as pl
from jax.experimental.pallas import tpu as pltpu
from jax.experimental.pallas import tpu_sc as plsc

_NUM_CARDS = 52
_NUM_RANKS = 13
_NUM_SUITS = 4
_HALF = 64
_EMBED = 128
_BLOCK = 8192


def _build_card_feats(rank_embed, suit_embed):
    nc = 1
    mesh = plsc.VectorSubcoreMesh(
        core_axis_name="c", subcore_axis_name="s", num_cores=nc)

    @functools.partial(
        pl.kernel,
        mesh=mesh,
        out_type=jax.ShapeDtypeStruct((_NUM_CARDS, _EMBED), jnp.float32),
        scratch_types=[
            pltpu.VMEM((_HALF,), jnp.float32),
            pltpu.VMEM((_NUM_SUITS, _HALF), jnp.float32),
            pltpu.VMEM((_NUM_SUITS, _EMBED), jnp.float32),
            pltpu.SemaphoreType.DMA,
            pltpu.SemaphoreType.DMA,
        ],
    )
    def build(rank_hbm, suit_hbm, out_hbm, rank_row, suit_v, out_v, sem1, sem2):
        wid = lax.axis_index("s") * nc + lax.axis_index("c")

        @pl.when(wid < _NUM_RANKS)
        def _():
            cp1 = pltpu.async_copy(rank_hbm.at[wid], rank_row, sem1)
            cp2 = pltpu.async_copy(suit_hbm, suit_v, sem2)
            cp1.wait()
            cp2.wait()
            for s in range(_NUM_SUITS):
                for j in range(_HALF // 16):
                    out_v[s, pl.ds(j * 16, 16)] = rank_row[pl.ds(j * 16, 16)]
                for j in range(_HALF // 16):
                    out_v[s, pl.ds(_HALF + j * 16, 16)] = suit_v[s, pl.ds(j * 16, 16)]
            pltpu.sync_copy(out_v, out_hbm.at[pl.ds(wid * _NUM_SUITS, _NUM_SUITS)])

    return build(rank_embed, suit_embed)


def _pool_body(hand_ref, cf_ref, out_ref):
    h = hand_ref[...]
    cf = cf_ref[...]
    row_r = lax.broadcasted_iota(jnp.int32, (_NUM_RANKS, _NUM_CARDS), 0)
    col_r = lax.broadcasted_iota(jnp.int32, (_NUM_RANKS, _NUM_CARDS), 1)
    sel_r = (col_r == 4 * row_r).astype(jnp.float32)
    row_s = lax.broadcasted_iota(jnp.int32, (_NUM_SUITS, _NUM_CARDS), 0)
    col_s = lax.broadcasted_iota(jnp.int32, (_NUM_SUITS, _NUM_CARDS), 1)
    sel_s = (col_s == row_s).astype(jnp.float32)
    rank_tab = jnp.dot(sel_r, cf[:, :_HALF], preferred_element_type=jnp.float32)
    suit_tab = jnp.dot(sel_s, cf[:, _HALF:], preferred_element_type=jnp.float32)
    gr = (col_r.T // 4 == row_r.T).astype(jnp.float32)
    gs = (col_s.T % 4 == row_s.T).astype(jnp.float32)
    hr = jnp.dot(h, gr, preferred_element_type=jnp.float32)
    hs = jnp.dot(h, gs, preferred_element_type=jnp.float32)
    cnt = jnp.maximum(jnp.sum(h, axis=1, keepdims=True), 1.0)
    out = jnp.concatenate(
        [jnp.dot(hr, rank_tab, preferred_element_type=jnp.float32),
         jnp.dot(hs, suit_tab, preferred_element_type=jnp.float32)], axis=1)
    out_ref[...] = out / cnt


def _pool(hand_onehot, card_feats):
    b = hand_onehot.shape[0]
    blk = _BLOCK if b % _BLOCK == 0 else b
    return pl.pallas_call(
        _pool_body,
        grid=(b // blk,),
        in_specs=[
            pl.BlockSpec((blk, _NUM_CARDS), lambda i: (i, 0)),
            pl.BlockSpec((_NUM_CARDS, _EMBED), lambda i: (0, 0)),
        ],
        out_specs=pl.BlockSpec((blk, _EMBED), lambda i: (i, 0)),
        out_shape=jax.ShapeDtypeStruct((b, _EMBED), jnp.float32),
    )(hand_onehot, card_feats)


def kernel(hand_onehot, rank_embed, suit_embed):
    card_feats = _build_card_feats(rank_embed, suit_embed)
    return _pool(hand_onehot, card_feats)


# speedup vs baseline: 1.0914x; 1.0914x over previous
import functools

import jax
import jax.numpy as jnp
from jax import lax
from jax.experimental import pallas as pl
from jax.experimental.pallas import tpu as pltpu
from jax.experimental.pallas import tpu_sc as plsc

_NUM_CARDS = 52
_NUM_RANKS = 13
_NUM_SUITS = 4
_HALF = 64
_EMBED = 128
_BLOCK = 8192


def _build_card_feats(rank_embed, suit_embed):
    nc = 1
    mesh = plsc.VectorSubcoreMesh(
        core_axis_name="c", subcore_axis_name="s", num_cores=nc)

    @functools.partial(
        pl.kernel,
        mesh=mesh,
        out_type=jax.ShapeDtypeStruct((_NUM_CARDS, _EMBED), jnp.float32),
        scratch_types=[
            pltpu.VMEM((_HALF,), jnp.float32),
            pltpu.VMEM((_NUM_SUITS, _HALF), jnp.float32),
            pltpu.VMEM((_NUM_SUITS, _EMBED), jnp.float32),
            pltpu.SemaphoreType.DMA,
            pltpu.SemaphoreType.DMA,
        ],
    )
    def build(rank_hbm, suit_hbm, out_hbm, rank_row, suit_v, out_v, sem1, sem2):
        wid = lax.axis_index("s") * nc + lax.axis_index("c")

        @pl.when(wid < _NUM_RANKS)
        def _():
            cp1 = pltpu.async_copy(rank_hbm.at[wid], rank_row, sem1)
            cp2 = pltpu.async_copy(suit_hbm, suit_v, sem2)
            cp1.wait()
            cp2.wait()
            for s in range(_NUM_SUITS):
                for j in range(_HALF // 16):
                    out_v[s, pl.ds(j * 16, 16)] = rank_row[pl.ds(j * 16, 16)]
                for j in range(_HALF // 16):
                    out_v[s, pl.ds(_HALF + j * 16, 16)] = suit_v[s, pl.ds(j * 16, 16)]
            pltpu.sync_copy(out_v, out_hbm.at[pl.ds(wid * _NUM_SUITS, _NUM_SUITS)])

    return build(rank_embed, suit_embed)


def _pool_body(hand_ref, cf_ref, out_ref):
    h = hand_ref[...]
    cf = cf_ref[...]
    cnt = jnp.maximum(jnp.sum(h, axis=1, keepdims=True), 1.0)
    out_ref[...] = jnp.dot(h, cf, preferred_element_type=jnp.float32) / cnt


def _pool(hand_onehot, card_feats):
    b = hand_onehot.shape[0]
    blk = _BLOCK if b % _BLOCK == 0 else b
    return pl.pallas_call(
        _pool_body,
        grid=(b // blk,),
        in_specs=[
            pl.BlockSpec((blk, _NUM_CARDS), lambda i: (i, 0)),
            pl.BlockSpec((_NUM_CARDS, _EMBED), lambda i: (0, 0)),
        ],
        out_specs=pl.BlockSpec((blk, _EMBED), lambda i: (i, 0)),
        out_shape=jax.ShapeDtypeStruct((b, _EMBED), jnp.float32),
    )(hand_onehot, card_feats)


def kernel(hand_onehot, rank_embed, suit_embed):
    card_feats = _build_card_feats(rank_embed, suit_embed)
    return _pool(hand_onehot, card_feats)
